# baseline (speedup 1.0000x reference)
import dataclasses
import functools

import jax
import jax.numpy as jnp
from jax import lax
from jax.experimental import pallas as pl
from jax.experimental.pallas import tpu as pltpu
from jax.experimental.pallas import tpu_sc as plsc

L = 16
NC = 2
NS = 16
NW = NC * NS

NEG_BIG = -3e38


def _sc_compiler_params():
    cp = pltpu.CompilerParams(use_tc_tiling_on_sc=False)
    if "needs_layout_passes" in pltpu.CompilerParams.__dataclass_fields__:
        cp = dataclasses.replace(cp, needs_layout_passes=False)
    return cp


def _dense_body(out_dim, feat_ref, wl_ref, bl_ref, wa_ref, ba_ref, mw_ref,
                haug_ref, as_ref, ad_ref):
    n = feat_ref.shape[0]
    mw = mw_ref[...]
    fm = feat_ref[...] * jax.nn.sigmoid(mw)[None, :]
    h = lax.dot_general(
        fm, wl_ref[...], (((1,), (1,)), ((), ())),
        precision=lax.Precision.HIGHEST,
        preferred_element_type=jnp.float32,
    ) + bl_ref[...][None, :]
    wa = wa_ref[...]
    a12 = jnp.concatenate([wa[:, :out_dim], wa[:, out_dim:]], axis=0)
    al = lax.dot_general(
        h, a12, (((1,), (1,)), ((), ())),
        precision=lax.Precision.HIGHEST,
        preferred_element_type=jnp.float32)
    as_ref[...] = al[:, 0:1]
    ad_ref[...] = al[:, 1:2] + ba_ref[...][0]
    haug_ref[:n, :out_dim] = h
    col = lax.broadcasted_iota(jnp.int32, (haug_ref.shape[0], L), 1)
    haug_ref[:, out_dim:] = jnp.where(col == 0, jnp.float32(1.0), jnp.float32(0.0))


def _score_body(n_nodes, ept, ch1, src_hbm, dst_hbm, as_hbm, ad_hbm,
                s_hbm, tmax_hbm, asv, adv, srcv, dstv, sv, mxv):
    cid = lax.axis_index("c")
    sid = lax.axis_index("s")
    wid = sid * NC + cid
    pltpu.sync_copy(as_hbm, asv)
    pltpu.sync_copy(ad_hbm, adv)
    mxv[...] = jnp.full((L,), NEG_BIG, jnp.float32)
    base0 = wid * ept

    @pl.loop(0, ept // ch1)
    def _chunk(c):
        base = base0 + c * ch1
        pltpu.sync_copy(src_hbm.at[pl.ds(base, ch1)], srcv)
        pltpu.sync_copy(dst_hbm.at[pl.ds(base, ch1)], dstv)

        @pl.loop(0, ch1 // L)
        def _group(g):
            i16 = srcv[pl.ds(g * L, L)]
            j16 = dstv[pl.ds(g * L, L)]
            a = plsc.load_gather(asv, [i16])
            b = plsc.load_gather(adv, [j16])
            s16 = a + b
            s16 = jnp.where(s16 >= 0, s16, s16 * jnp.float32(0.01))
            sv[pl.ds(g * L, L)] = s16
            mxv[...] = jnp.maximum(mxv[...], s16)

        pltpu.sync_copy(sv, s_hbm.at[pl.ds(base, ch1)])

    pltpu.sync_copy(mxv, tmax_hbm.at[wid])


def _aggregate_body(np_pad, da, ch2, nsup, jrows,
                    src2_hbm, dst2_hbm, s2_hbm, tmax_hbm, haug_hbm, zeros_hbm,
                    part_hbm,
                    srcb, dstb, sb, rows0, rows1, mxall, exb, acc,
                    gsem0, gsem1, ssem0, ssem1):
    cid = lax.axis_index("c")
    sid = lax.axis_index("s")
    wid = sid * NC + cid
    rows_per_sub = np_pad // NS
    rowbase0 = wid * nsup * jrows

    def splat(vec, r):
        return lax.gather(
            vec, jnp.full((L, 1), r, jnp.int32),
            dimension_numbers=lax.GatherDimensionNumbers(
                offset_dims=(), collapsed_slice_dims=(0,),
                start_index_map=(0,)),
            slice_sizes=(1,),
            mode=lax.GatherScatterMode.PROMISE_IN_BOUNDS)

    def gather_start(c, rbuf, sem):
        pltpu.async_copy(haug_hbm.at[srcb.at[c]], rbuf, sem)

    def gather_wait(c, rbuf, sem):
        pltpu.make_async_copy(haug_hbm.at[srcb.at[c]], rbuf, sem).wait()

    def scatter_start(c, rbuf, sem):
        pltpu.async_copy(rbuf, acc.at[dstb.at[c]], sem, add=True)

    def scatter_wait(c, rbuf, sem):
        pltpu.make_async_copy(rbuf, acc.at[dstb.at[c]], sem).wait()

    def scale(c, rbuf, gmax):
        @pl.loop(0, ch2 // L)
        def _group(g):
            s16 = sb[c, pl.ds(g * L, L)]
            exb[pl.ds(g * L, L)] = jnp.exp(s16 - gmax)

        @plsc.parallel_loop(0, ch2, unroll=4)
        def _row(r):
            es = plsc.load_gather(exb, [jnp.full((L,), r, jnp.int32)])
            for k in range(da // L - 1):
                rbuf[r, pl.ds(k * L, L)] = rbuf[r, pl.ds(k * L, L)] * es
            rbuf[r, pl.ds(da - L, L)] = es

    pltpu.sync_copy(tmax_hbm, mxall)
    m16 = mxall[0]
    for i in range(1, NW):
        m16 = jnp.maximum(m16, mxall[i])
    gmax = jnp.max(m16)

    pltpu.sync_copy(zeros_hbm, acc.at[pl.ds(sid * rows_per_sub, rows_per_sub)])

    plsc.subcore_barrier()

    @pl.loop(0, nsup)
    def _block(sblk):
        rb = rowbase0 + sblk * jrows
        pltpu.sync_copy(src2_hbm.at[pl.ds(rb, jrows)], srcb)
        pltpu.sync_copy(dst2_hbm.at[pl.ds(rb, jrows)], dstb)
        pltpu.sync_copy(s2_hbm.at[pl.ds(rb, jrows)], sb)
        gather_start(0, rows0, gsem0)

        @pl.loop(0, (jrows - 1) // 2)
        def _pair(t):
            a = 2 * t
            b = a + 1
            gather_wait(a, rows0, gsem0)

            @pl.when(t > 0)
            def _():
                scatter_wait(b - 2, rows1, ssem1)

            gather_start(b, rows1, gsem1)
            scale(a, rows0, gmax)
            scatter_start(a, rows0, ssem0)
            gather_wait(b, rows1, gsem1)
            scatter_wait(a, rows0, ssem0)
            gather_start(a + 2, rows0, gsem0)
            scale(b, rows1, gmax)
            scatter_start(b, rows1, ssem1)

        last = jrows - 1
        gather_wait(last, rows0, gsem0)
        scatter_wait(last - 1, rows1, ssem1)
        scale(last, rows0, gmax)
        pltpu.sync_copy(rows0, acc.at[dstb.at[last]], add=True)

    plsc.subcore_barrier()
    pltpu.sync_copy(acc.at[pl.ds(sid * rows_per_sub, rows_per_sub)],
                    part_hbm.at[cid, pl.ds(sid * rows_per_sub, rows_per_sub)])


def _combine_body(np_pad, da, out_dim, bs, part_hbm, out_hbm, b0, b1, ob):
    cid = lax.axis_index("c")
    sid = lax.axis_index("s")
    wid = sid * NC + cid
    rpt = np_pad // NW
    base = wid * rpt

    @pl.loop(0, rpt // bs)
    def _blk(z):
        off = base + z * bs
        pltpu.sync_copy(part_hbm.at[0, pl.ds(off, bs)], b0)
        pltpu.sync_copy(part_hbm.at[1, pl.ds(off, bs)], b1)

        @plsc.parallel_loop(0, bs, unroll=2)
        def _row(r):
            d16 = b0[r, pl.ds(out_dim, L)] + b1[r, pl.ds(out_dim, L)]
            den = lax.gather(
                d16, jnp.zeros((L, 1), jnp.int32),
                dimension_numbers=lax.GatherDimensionNumbers(
                    offset_dims=(), collapsed_slice_dims=(0,),
                    start_index_map=(0,)),
                slice_sizes=(1,),
                mode=lax.GatherScatterMode.PROMISE_IN_BOUNDS)
            den = jnp.maximum(den, jnp.float32(1e-9))
            for k in range(out_dim // L):
                ob[r, pl.ds(k * L, L)] = (
                    b0[r, pl.ds(k * L, L)] + b1[r, pl.ds(k * L, L)]) / den

        pltpu.sync_copy(ob, out_hbm.at[pl.ds(off, bs)])


@jax.jit
def _run(feat, edge_index, W_lin, b_lin, W_attn, b_attn, mask_w):
    n, in_dim = feat.shape
    out_dim = W_lin.shape[0]
    e = edge_index.shape[1]
    da = out_dim + L
    ept = e // NW
    ch1 = 2000
    ch2 = 80
    jrows = 25
    nsup = ept // (ch2 * jrows)
    rps = -(-n // NS)
    rps = -(-rps // 32) * 32
    np_pad = rps * NS
    assert ept % ch1 == 0 and ch1 % L == 0
    assert ept % (ch2 * jrows) == 0 and ch2 % L == 0 and ch2 <= 128
    assert jrows % 2 == 1

    haug, alpha_s, alpha_d = pl.pallas_call(
        functools.partial(_dense_body, out_dim),
        out_shape=[
            jax.ShapeDtypeStruct((np_pad, da), jnp.float32),
            jax.ShapeDtypeStruct((n, 1), jnp.float32),
            jax.ShapeDtypeStruct((n, 1), jnp.float32),
        ],
    )(feat, W_lin, b_lin, W_attn, b_attn, mask_w)
    alpha_s = alpha_s.reshape(n)
    alpha_d = alpha_d.reshape(n)

    src = edge_index[0]
    dst = edge_index[1]

    mesh = plsc.VectorSubcoreMesh(core_axis_name="c", subcore_axis_name="s",
                                  num_cores=NC, num_subcores=NS)

    score_kernel = pl.kernel(
        functools.partial(_score_body, n, ept, ch1),
        out_type=[
            jax.ShapeDtypeStruct((e,), jnp.float32),
            jax.ShapeDtypeStruct((NW, L), jnp.float32),
        ],
        mesh=mesh,
        scratch_types=[
            pltpu.VMEM((n,), jnp.float32),
            pltpu.VMEM((n,), jnp.float32),
            pltpu.VMEM((ch1,), jnp.int32),
            pltpu.VMEM((ch1,), jnp.int32),
            pltpu.VMEM((ch1,), jnp.float32),
            pltpu.VMEM((L,), jnp.float32),
        ],
        compiler_params=_sc_compiler_params(),
    )
    s, tmax = score_kernel(src, dst, alpha_s, alpha_d)

    src2 = src.reshape(e // ch2, ch2)
    dst2 = dst.reshape(e // ch2, ch2)
    s2 = s.reshape(e // ch2, ch2)

    agg_kernel = pl.kernel(
        functools.partial(_aggregate_body, np_pad, da, ch2, nsup, jrows),
        out_type=jax.ShapeDtypeStruct((NC, np_pad, da), jnp.float32),
        mesh=mesh,
        scratch_types=[
            pltpu.VMEM((jrows, ch2), jnp.int32),
            pltpu.VMEM((jrows, ch2), jnp.int32),
            pltpu.VMEM((jrows, ch2), jnp.float32),
            pltpu.VMEM((ch2, da), jnp.float32),
            pltpu.VMEM((ch2, da), jnp.float32),
            pltpu.VMEM((NW, L), jnp.float32),
            pltpu.VMEM((ch2,), jnp.float32),
            pltpu.VMEM_SHARED((np_pad, da), jnp.float32),
            pltpu.SemaphoreType.DMA,
            pltpu.SemaphoreType.DMA,
            pltpu.SemaphoreType.DMA,
            pltpu.SemaphoreType.DMA,
        ],
        compiler_params=_sc_compiler_params(),
    )
    zeros = jnp.zeros((np_pad // NS, da), jnp.float32)
    part = agg_kernel(src2, dst2, s2, tmax, haug, zeros)

    bs = 64
    assert (np_pad // NW) % bs == 0
    combine_kernel = pl.kernel(
        functools.partial(_combine_body, np_pad, da, out_dim, bs),
        out_type=jax.ShapeDtypeStruct((np_pad, out_dim), jnp.float32),
        mesh=mesh,
        scratch_types=[
            pltpu.VMEM((bs, da), jnp.float32),
            pltpu.VMEM((bs, da), jnp.float32),
            pltpu.VMEM((bs, out_dim), jnp.float32),
        ],
        compiler_params=_sc_compiler_params(),
    )
    h_new = combine_kernel(part)
    return h_new[:n]


def kernel(feat, edge_index, W_lin, b_lin, W_attn, b_attn, mask_w):
    return _run(feat, edge_index, W_lin, b_lin, W_attn, b_attn, mask_w)


# speedup vs baseline: 1.0304x; 1.0304x over previous
import dataclasses
import functools

import jax
import jax.numpy as jnp
from jax import lax
from jax.experimental import pallas as pl
from jax.experimental.pallas import tpu as pltpu
from jax.experimental.pallas import tpu_sc as plsc

L = 16
NC = 2
NS = 16
NW = NC * NS

NEG_BIG = -3e38


def _sc_compiler_params():
    cp = pltpu.CompilerParams(use_tc_tiling_on_sc=False)
    if "needs_layout_passes" in pltpu.CompilerParams.__dataclass_fields__:
        cp = dataclasses.replace(cp, needs_layout_passes=False)
    return cp


def _dense_body(out_dim, feat_ref, wl_ref, bl_ref, wa_ref, ba_ref, mw_ref,
                haug_ref, as_ref, ad_ref):
    n = feat_ref.shape[0]
    mw = mw_ref[...]
    fm = feat_ref[...] * jax.nn.sigmoid(mw)[None, :]
    h = lax.dot_general(
        fm, wl_ref[...], (((1,), (1,)), ((), ())),
        precision=lax.Precision.HIGHEST,
        preferred_element_type=jnp.float32,
    ) + bl_ref[...][None, :]
    wa = wa_ref[...]
    a1 = wa[0, :out_dim]
    a2 = wa[0, out_dim:]
    as_ref[...] = jnp.sum(h * a1[None, :], axis=1, keepdims=True)
    ad_ref[...] = jnp.sum(h * a2[None, :], axis=1, keepdims=True) + ba_ref[...][0]
    haug_ref[:n, :out_dim] = h
    col = lax.broadcasted_iota(jnp.int32, (haug_ref.shape[0], L), 1)
    haug_ref[:, out_dim:] = jnp.where(col == 0, jnp.float32(1.0), jnp.float32(0.0))


def _score_body(n_nodes, ept, ch1, src_hbm, dst_hbm, as_hbm, ad_hbm,
                s_hbm, tmax_hbm, asv, adv, srcv, dstv, sv, mxv):
    cid = lax.axis_index("c")
    sid = lax.axis_index("s")
    wid = sid * NC + cid
    pltpu.sync_copy(as_hbm, asv)
    pltpu.sync_copy(ad_hbm, adv)
    mxv[...] = jnp.full((L,), NEG_BIG, jnp.float32)
    base0 = wid * ept

    @pl.loop(0, ept // ch1)
    def _chunk(c):
        base = base0 + c * ch1
        pltpu.sync_copy(src_hbm.at[pl.ds(base, ch1)], srcv)
        pltpu.sync_copy(dst_hbm.at[pl.ds(base, ch1)], dstv)

        @pl.loop(0, ch1 // L)
        def _group(g):
            i16 = srcv[pl.ds(g * L, L)]
            j16 = dstv[pl.ds(g * L, L)]
            a = plsc.load_gather(asv, [i16])
            b = plsc.load_gather(adv, [j16])
            s16 = a + b
            s16 = jnp.where(s16 >= 0, s16, s16 * jnp.float32(0.01))
            sv[pl.ds(g * L, L)] = s16
            mxv[...] = jnp.maximum(mxv[...], s16)

        pltpu.sync_copy(sv, s_hbm.at[pl.ds(base, ch1)])

    pltpu.sync_copy(mxv, tmax_hbm.at[wid])


def _aggregate_body(np_pad, da, ch2, nsup, jrows,
                    src2_hbm, dst2_hbm, s2_hbm, tmax_hbm, haug_hbm, zeros_hbm,
                    part_hbm,
                    srcb, dstb, sb, rows0, rows1, mxall, exb, acc,
                    gsem0, gsem1, ssem0, ssem1):
    cid = lax.axis_index("c")
    sid = lax.axis_index("s")
    wid = sid * NC + cid
    rows_per_sub = np_pad // NS
    rowbase0 = wid * nsup * jrows

    def splat(vec, r):
        return lax.gather(
            vec, jnp.full((L, 1), r, jnp.int32),
            dimension_numbers=lax.GatherDimensionNumbers(
                offset_dims=(), collapsed_slice_dims=(0,),
                start_index_map=(0,)),
            slice_sizes=(1,),
            mode=lax.GatherScatterMode.PROMISE_IN_BOUNDS)

    def gather_start(c, rbuf, sem):
        pltpu.async_copy(haug_hbm.at[srcb.at[c]], rbuf, sem)

    def gather_wait(c, rbuf, sem):
        pltpu.make_async_copy(haug_hbm.at[srcb.at[c]], rbuf, sem).wait()

    def scatter_start(c, rbuf, sem):
        pltpu.async_copy(rbuf, acc.at[dstb.at[c]], sem, add=True)

    def scatter_wait(c, rbuf, sem):
        pltpu.make_async_copy(rbuf, acc.at[dstb.at[c]], sem).wait()

    def scale(c, rbuf, gmax):
        @pl.loop(0, ch2 // L)
        def _group(g):
            s16 = sb[c, pl.ds(g * L, L)]
            exb[pl.ds(g * L, L)] = jnp.exp(s16 - gmax)

        @plsc.parallel_loop(0, ch2, unroll=4)
        def _row(r):
            es = plsc.load_gather(exb, [jnp.full((L,), r, jnp.int32)])
            for k in range(da // L - 1):
                rbuf[r, pl.ds(k * L, L)] = rbuf[r, pl.ds(k * L, L)] * es
            rbuf[r, pl.ds(da - L, L)] = es

    pltpu.sync_copy(tmax_hbm, mxall)
    m16 = mxall[0]
    for i in range(1, NW):
        m16 = jnp.maximum(m16, mxall[i])
    gmax = jnp.max(m16)

    pltpu.sync_copy(zeros_hbm, acc.at[pl.ds(sid * rows_per_sub, rows_per_sub)])

    plsc.subcore_barrier()

    @pl.loop(0, nsup)
    def _block(sblk):
        rb = rowbase0 + sblk * jrows
        pltpu.sync_copy(src2_hbm.at[pl.ds(rb, jrows)], srcb)
        pltpu.sync_copy(dst2_hbm.at[pl.ds(rb, jrows)], dstb)
        pltpu.sync_copy(s2_hbm.at[pl.ds(rb, jrows)], sb)
        gather_start(0, rows0, gsem0)

        @pl.loop(0, (jrows - 1) // 2)
        def _pair(t):
            a = 2 * t
            b = a + 1
            gather_wait(a, rows0, gsem0)

            @pl.when(t > 0)
            def _():
                scatter_wait(b - 2, rows1, ssem1)

            gather_start(b, rows1, gsem1)
            scale(a, rows0, gmax)
            scatter_start(a, rows0, ssem0)
            gather_wait(b, rows1, gsem1)
            scatter_wait(a, rows0, ssem0)
            gather_start(a + 2, rows0, gsem0)
            scale(b, rows1, gmax)
            scatter_start(b, rows1, ssem1)

        last = jrows - 1
        gather_wait(last, rows0, gsem0)
        scatter_wait(last - 1, rows1, ssem1)
        scale(last, rows0, gmax)
        pltpu.sync_copy(rows0, acc.at[dstb.at[last]], add=True)

    plsc.subcore_barrier()
    pltpu.sync_copy(acc.at[pl.ds(sid * rows_per_sub, rows_per_sub)],
                    part_hbm.at[cid, pl.ds(sid * rows_per_sub, rows_per_sub)])


def _combine_body(np_pad, da, out_dim, bs, part_hbm, out_hbm, b0, b1, ob):
    cid = lax.axis_index("c")
    sid = lax.axis_index("s")
    wid = sid * NC + cid
    rpt = np_pad // NW
    base = wid * rpt

    @pl.loop(0, rpt // bs)
    def _blk(z):
        off = base + z * bs
        pltpu.sync_copy(part_hbm.at[0, pl.ds(off, bs)], b0)
        pltpu.sync_copy(part_hbm.at[1, pl.ds(off, bs)], b1)

        @plsc.parallel_loop(0, bs, unroll=2)
        def _row(r):
            d16 = b0[r, pl.ds(out_dim, L)] + b1[r, pl.ds(out_dim, L)]
            den = lax.gather(
                d16, jnp.zeros((L, 1), jnp.int32),
                dimension_numbers=lax.GatherDimensionNumbers(
                    offset_dims=(), collapsed_slice_dims=(0,),
                    start_index_map=(0,)),
                slice_sizes=(1,),
                mode=lax.GatherScatterMode.PROMISE_IN_BOUNDS)
            den = jnp.maximum(den, jnp.float32(1e-9))
            for k in range(out_dim // L):
                ob[r, pl.ds(k * L, L)] = (
                    b0[r, pl.ds(k * L, L)] + b1[r, pl.ds(k * L, L)]) / den

        pltpu.sync_copy(ob, out_hbm.at[pl.ds(off, bs)])


@jax.jit
def _run(feat, edge_index, W_lin, b_lin, W_attn, b_attn, mask_w):
    n, in_dim = feat.shape
    out_dim = W_lin.shape[0]
    e = edge_index.shape[1]
    da = out_dim + L
    ept = e // NW
    ch1 = 2000
    ch2 = 80
    jrows = 25
    nsup = ept // (ch2 * jrows)
    rps = -(-n // NS)
    rps = -(-rps // 32) * 32
    np_pad = rps * NS
    assert ept % ch1 == 0 and ch1 % L == 0
    assert ept % (ch2 * jrows) == 0 and ch2 % L == 0 and ch2 <= 128
    assert jrows % 2 == 1

    haug, alpha_s, alpha_d = pl.pallas_call(
        functools.partial(_dense_body, out_dim),
        out_shape=[
            jax.ShapeDtypeStruct((np_pad, da), jnp.float32),
            jax.ShapeDtypeStruct((n, 1), jnp.float32),
            jax.ShapeDtypeStruct((n, 1), jnp.float32),
        ],
    )(feat, W_lin, b_lin, W_attn, b_attn, mask_w)
    alpha_s = alpha_s.reshape(n)
    alpha_d = alpha_d.reshape(n)

    src = edge_index[0]
    dst = edge_index[1]

    mesh = plsc.VectorSubcoreMesh(core_axis_name="c", subcore_axis_name="s",
                                  num_cores=NC, num_subcores=NS)

    score_kernel = pl.kernel(
        functools.partial(_score_body, n, ept, ch1),
        out_type=[
            jax.ShapeDtypeStruct((e,), jnp.float32),
            jax.ShapeDtypeStruct((NW, L), jnp.float32),
        ],
        mesh=mesh,
        scratch_types=[
            pltpu.VMEM((n,), jnp.float32),
            pltpu.VMEM((n,), jnp.float32),
            pltpu.VMEM((ch1,), jnp.int32),
            pltpu.VMEM((ch1,), jnp.int32),
            pltpu.VMEM((ch1,), jnp.float32),
            pltpu.VMEM((L,), jnp.float32),
        ],
        compiler_params=_sc_compiler_params(),
    )
    s, tmax = score_kernel(src, dst, alpha_s, alpha_d)

    src2 = src.reshape(e // ch2, ch2)
    dst2 = dst.reshape(e // ch2, ch2)
    s2 = s.reshape(e // ch2, ch2)

    agg_kernel = pl.kernel(
        functools.partial(_aggregate_body, np_pad, da, ch2, nsup, jrows),
        out_type=jax.ShapeDtypeStruct((NC, np_pad, da), jnp.float32),
        mesh=mesh,
        scratch_types=[
            pltpu.VMEM((jrows, ch2), jnp.int32),
            pltpu.VMEM((jrows, ch2), jnp.int32),
            pltpu.VMEM((jrows, ch2), jnp.float32),
            pltpu.VMEM((ch2, da), jnp.float32),
            pltpu.VMEM((ch2, da), jnp.float32),
            pltpu.VMEM((NW, L), jnp.float32),
            pltpu.VMEM((ch2,), jnp.float32),
            pltpu.VMEM_SHARED((np_pad, da), jnp.float32),
            pltpu.SemaphoreType.DMA,
            pltpu.SemaphoreType.DMA,
            pltpu.SemaphoreType.DMA,
            pltpu.SemaphoreType.DMA,
        ],
        compiler_params=_sc_compiler_params(),
    )
    zeros = jnp.zeros((np_pad // NS, da), jnp.float32)
    part = agg_kernel(src2, dst2, s2, tmax, haug, zeros)

    bs = 64
    assert (np_pad // NW) % bs == 0
    combine_kernel = pl.kernel(
        functools.partial(_combine_body, np_pad, da, out_dim, bs),
        out_type=jax.ShapeDtypeStruct((np_pad, out_dim), jnp.float32),
        mesh=mesh,
        scratch_types=[
            pltpu.VMEM((bs, da), jnp.float32),
            pltpu.VMEM((bs, da), jnp.float32),
            pltpu.VMEM((bs, out_dim), jnp.float32),
        ],
        compiler_params=_sc_compiler_params(),
    )
    h_new = combine_kernel(part)
    return h_new[:n]


def kernel(feat, edge_index, W_lin, b_lin, W_attn, b_attn, mask_w):
    return _run(feat, edge_index, W_lin, b_lin, W_attn, b_attn, mask_w)
